# baseline (speedup 1.0000x reference)
import dataclasses
import functools

import jax
import jax.numpy as jnp
from jax import lax
from jax.experimental import pallas as pl
from jax.experimental.pallas import tpu as pltpu
from jax.experimental.pallas import tpu_sc as plsc

N = 10000
NP = 10240
E = 320000
D = 128
DG = 64
DDEN = 16
NSUB = 16
EPT = E // NSUB
C = 80
EPTP = 20160
NCH = EPTP // C
RPT = NP // NSUB
LANES = 16
NBUF = 6


def _tc_project(x, W, attn_l, attn_r):
    def body(x_ref, w_ref, al_ref, ar_ref, feat2_ref, el_ref, er_ref):
        feat = jnp.dot(x_ref[...], w_ref[...],
                       preferred_element_type=jnp.float32)
        fb = feat.astype(jnp.bfloat16)
        feat2_ref[0, :N, :] = fb[:, :DG]
        feat2_ref[1, :N, :] = fb[:, DG:]
        el_ref[pl.ds(0, N)] = jnp.sum(feat * al_ref[...][None, :], axis=1)
        er_ref[pl.ds(0, N)] = jnp.sum(feat * ar_ref[...][None, :], axis=1)
        zpad = jnp.zeros((NP - N, DG), jnp.bfloat16)
        feat2_ref[0, pl.ds(N, NP - N), :] = zpad
        feat2_ref[1, pl.ds(N, NP - N), :] = zpad
        el_ref[pl.ds(N, NP - N)] = jnp.zeros((NP - N,), jnp.float32)
        er_ref[pl.ds(N, NP - N)] = jnp.zeros((NP - N,), jnp.float32)

    return pl.pallas_call(
        body,
        out_shape=(
            jax.ShapeDtypeStruct((2, NP, DG), jnp.bfloat16),
            jax.ShapeDtypeStruct((NP,), jnp.float32),
            jax.ShapeDtypeStruct((NP,), jnp.float32),
        ),
    )(x, W, attn_l, attn_r)


FB = 80


def _sc_edge_aggregate(feat2, el, er, idx4, z64, z16, bias):
    mesh = plsc.VectorSubcoreMesh(core_axis_name="c", subcore_axis_name="s")
    cp = pltpu.CompilerParams()
    if "needs_layout_passes" in pltpu.CompilerParams.__dataclass_fields__:
        cp = dataclasses.replace(cp, needs_layout_passes=False)
    if "use_tc_tiling_on_sc" in pltpu.CompilerParams.__dataclass_fields__:
        cp = dataclasses.replace(cp, use_tc_tiling_on_sc=False)

    @functools.partial(
        pl.kernel,
        compiler_params=cp,
        out_type=jax.ShapeDtypeStruct((2, NP, DG), jnp.float32),
        mesh=mesh,
        scratch_types=(
            [
                pltpu.VMEM((NP,), jnp.float32),
                pltpu.VMEM((NP,), jnp.float32),
                pltpu.VMEM((D,), jnp.float32),
            ]
            + [pltpu.VMEM((2, C), jnp.int32)] * NBUF
            + [pltpu.VMEM((C, DG), jnp.bfloat16)] * NBUF
            + [pltpu.VMEM((C, DG), jnp.float32)] * NBUF
            + [pltpu.VMEM((C, DDEN), jnp.float32)] * NBUF
            + [
                pltpu.VMEM_SHARED((NP, DG), jnp.float32),
                pltpu.VMEM_SHARED((NP, DDEN), jnp.float32),
            ]
            + [pltpu.SemaphoreType.DMA] * (3 * NBUF + 1)
        ),
    )
    def k(feat2_hbm, el_hbm, er_hbm, idx_hbm, z64_hbm, z16_hbm, bias_hbm,
          out_hbm,
          el_v, er_v, bias_v, i0, i1, i2, i3, i4, i5, h0, h1, h2, h3, h4, h5,
          r0, r1, r2, r3, r4, r5,
          w0, w1, w2, w3, w4, w5, acc_sh, den_sh,
          a0, a1, a2, a3, a4, a5, g0, g1, g2, g3, g4, g5,
          s0, s1, s2, s3, s4, s5, stg):
        cid = lax.axis_index("c")
        sid = lax.axis_index("s")
        idxb = [i0, i1, i2, i3, i4, i5]
        rbf = [h0, h1, h2, h3, h4, h5]
        rows = [r0, r1, r2, r3, r4, r5]
        wden = [w0, w1, w2, w3, w4, w5]
        isem = [a0, a1, a2, a3, a4, a5]
        gsem = [g0, g1, g2, g3, g4, g5]
        ssem = [s0, s1, s2, s3, s4, s5]
        tile = pl.ds(sid * RPT, RPT)

        copies = [
            (z64_hbm.at[tile], acc_sh.at[tile]),
            (z16_hbm.at[tile], den_sh.at[tile]),
            (el_hbm, el_v),
            (er_hbm, er_v),
            (bias_hbm, bias_v),
        ]
        descs = [pltpu.async_copy(a, b, stg) for a, b in copies]
        for d in descs:
            d.wait()
        plsc.subcore_barrier()

        def idx_start(m, b):
            pltpu.async_copy(idx_hbm.at[sid, m], idxb[b], isem[b])

        def idx_wait(m, b):
            pltpu.make_async_copy(idx_hbm.at[sid, m], idxb[b], isem[b]).wait()

        def gather_start(b):
            pltpu.async_copy(feat2_hbm.at[cid].at[idxb[b].at[0]],
                             rbf[b], gsem[b])

        def gather_wait(b):
            pltpu.make_async_copy(feat2_hbm.at[cid].at[idxb[b].at[0]],
                                  rbf[b], gsem[b]).wait()

        def scat_start(b):
            pltpu.async_copy(rows[b], acc_sh.at[idxb[b].at[1]], ssem[b],
                             add=True)
            pltpu.async_copy(wden[b], den_sh.at[idxb[b].at[1]], ssem[b],
                             add=True)

        def scat_wait(b):
            pltpu.make_async_copy(rows[b], acc_sh.at[idxb[b].at[1]],
                                  ssem[b]).wait()
            pltpu.make_async_copy(wden[b], den_sh.at[idxb[b].at[1]],
                                  ssem[b]).wait()

        def process(b):
            @plsc.parallel_loop(0, C, step=LANES)
            def _grp(g):
                si = idxb[b][0, pl.ds(g, LANES)]
                di = idxb[b][1, pl.ds(g, LANES)]
                e = plsc.load_gather(el_v, [si]) + plsc.load_gather(er_v, [di])
                e = jnp.where(e >= 0.0, e, 0.2 * e)
                wv = jnp.exp(e)
                for rr in range(LANES):
                    wb = jnp.broadcast_to(wv[rr], (LANES,))
                    wden[b][g + rr, :] = wb
                    for q in range(DG // (2 * LANES)):
                        packed = rbf[b][g + rr, pl.ds(q * 2 * LANES, 2 * LANES)]
                        lo, hi = plsc.unpack(packed,
                                             format=plsc.PackFormat.INTERLEAVED)
                        base = q * 2 * LANES
                        rows[b][g + rr, pl.ds(base, LANES)] = lo * wb
                        rows[b][g + rr, pl.ds(base + LANES, LANES)] = hi * wb

        idx_start(0, 0)
        idx_start(1, 1)
        idx_start(2, 2)
        idx_wait(0, 0)
        gather_start(0)
        idx_wait(1, 1)
        gather_start(1)
        for s in range(NBUF):
            if s >= 3:
                scat_wait(s - 3)
            idx_start(s + 3, (s + 3) % NBUF)
            idx_wait(s + 2, (s + 2) % NBUF)
            gather_start((s + 2) % NBUF)
            gather_wait(s)
            process(s)
            scat_start(s)

        @pl.loop(NBUF, NCH, step=NBUF)
        def _steady(j):
            for off in range(NBUF):
                m = j + off
                bb3 = (off + 3) % NBUF
                bb2 = (off + 2) % NBUF
                scat_wait(bb3)

                @pl.when(m + 3 < NCH)
                def _():
                    idx_start(m + 3, bb3)

                @pl.when(m + 2 < NCH)
                def _():
                    idx_wait(m + 2, bb2)
                    gather_start(bb2)

                gather_wait(off)
                process(off)
                scat_start(off)

        scat_wait((NCH - 3) % NBUF)
        scat_wait((NCH - 2) % NBUF)
        scat_wait((NCH - 1) % NBUF)

        plsc.subcore_barrier()

        bias_regs = [bias_v[pl.ds(cid * DG + q * LANES, LANES)]
                     for q in range(DG // LANES)]

        @pl.loop(0, RPT, step=FB)
        def _fin(f):
            rb = sid * RPT + f
            pltpu.sync_copy(acc_sh.at[pl.ds(rb, FB)], r0.at[pl.ds(0, FB)])
            pltpu.sync_copy(den_sh.at[pl.ds(rb, FB)], w0.at[pl.ds(0, FB)])

            @pl.loop(0, FB)
            def _r(r):
                db = w0[r, pl.ds(0, LANES)]
                ok = db > 0.0
                for q in range(DG // LANES):
                    sl = pl.ds(q * LANES, LANES)
                    v = r0[r, sl]
                    r0[r, sl] = jnp.where(ok, v / db, 0.0) + bias_regs[q]

            pltpu.sync_copy(r0.at[pl.ds(0, FB)], out_hbm.at[cid, pl.ds(rb, FB)])

    return k(feat2, el, er, idx4, z64, z16, bias)


def kernel(x, edge_index, W, attn_l, attn_r, bias):
    feat2, el, er = _tc_project(x, W, attn_l, attn_r)
    feat2 = (feat2.reshape(2, NP, 2, 2, 16)
             .transpose(0, 1, 2, 4, 3)
             .reshape(2, NP, DG))
    ei3 = edge_index.reshape(2, NSUB, EPT)
    pad = jnp.full((2, NSUB, EPTP - EPT), N, jnp.int32)
    ei4 = jnp.concatenate([ei3, pad], axis=2)
    idx4 = ei4.reshape(2, NSUB, NCH, C).transpose(1, 2, 0, 3)
    z64 = jnp.zeros((NP, DG), jnp.float32)
    z16 = jnp.zeros((NP, DDEN), jnp.float32)
    halves = _sc_edge_aggregate(feat2, el, er, idx4, z64, z16, bias)
    return jnp.concatenate([halves[0, :N], halves[1, :N]], axis=1)


# speedup vs baseline: 1.3819x; 1.3819x over previous
import dataclasses
import functools

import jax
import jax.numpy as jnp
from jax import lax
from jax.experimental import pallas as pl
from jax.experimental.pallas import tpu as pltpu
from jax.experimental.pallas import tpu_sc as plsc

N = 10000
NP = 10240
E = 320000
D = 128
DG = 64
DDEN = 16
NSUB = 16
EPT = E // NSUB
C = 96
EPTP = 20160
NCH = EPTP // C
RPT = NP // NSUB
LANES = 16
NBUF = 6


def _tc_project(x, W, attn_l, attn_r):
    def body(x_ref, w_ref, al_ref, ar_ref, feat2_ref, el_ref, er_ref):
        feat = jnp.dot(x_ref[...], w_ref[...],
                       preferred_element_type=jnp.float32)
        feat2_ref[0, :N, :] = feat[:, :DG]
        feat2_ref[1, :N, :] = feat[:, DG:]
        el_ref[pl.ds(0, N)] = jnp.sum(feat * al_ref[...][None, :], axis=1)
        er_ref[pl.ds(0, N)] = jnp.sum(feat * ar_ref[...][None, :], axis=1)
        zpad = jnp.zeros((NP - N, DG), jnp.float32)
        feat2_ref[0, pl.ds(N, NP - N), :] = zpad
        feat2_ref[1, pl.ds(N, NP - N), :] = zpad
        el_ref[pl.ds(N, NP - N)] = jnp.zeros((NP - N,), jnp.float32)
        er_ref[pl.ds(N, NP - N)] = jnp.zeros((NP - N,), jnp.float32)

    return pl.pallas_call(
        body,
        out_shape=(
            jax.ShapeDtypeStruct((2, NP, DG), jnp.float32),
            jax.ShapeDtypeStruct((NP,), jnp.float32),
            jax.ShapeDtypeStruct((NP,), jnp.float32),
        ),
    )(x, W, attn_l, attn_r)


FB = 80


def _sc_edge_aggregate(feat2, el, er, idx4, z64, z16, bias):
    mesh = plsc.VectorSubcoreMesh(core_axis_name="c", subcore_axis_name="s")
    cp = pltpu.CompilerParams()
    if "needs_layout_passes" in pltpu.CompilerParams.__dataclass_fields__:
        cp = dataclasses.replace(cp, needs_layout_passes=False)
    if "use_tc_tiling_on_sc" in pltpu.CompilerParams.__dataclass_fields__:
        cp = dataclasses.replace(cp, use_tc_tiling_on_sc=False)

    @functools.partial(
        pl.kernel,
        compiler_params=cp,
        out_type=jax.ShapeDtypeStruct((2, NP, DG), jnp.float32),
        mesh=mesh,
        scratch_types=(
            [
                pltpu.VMEM((NP,), jnp.float32),
                pltpu.VMEM((NP,), jnp.float32),
                pltpu.VMEM((D,), jnp.float32),
            ]
            + [pltpu.VMEM((2, C), jnp.int32)] * NBUF
            + [pltpu.VMEM((C, DG), jnp.float32)] * NBUF
            + [pltpu.VMEM((C, DDEN), jnp.float32)] * NBUF
            + [
                pltpu.VMEM_SHARED((NP, DG), jnp.float32),
                pltpu.VMEM_SHARED((NP, DDEN), jnp.float32),
            ]
            + [pltpu.SemaphoreType.DMA] * (3 * NBUF + 1)
        ),
    )
    def k(feat2_hbm, el_hbm, er_hbm, idx_hbm, z64_hbm, z16_hbm, bias_hbm,
          out_hbm,
          el_v, er_v, bias_v, i0, i1, i2, i3, i4, i5, r0, r1, r2, r3, r4, r5,
          w0, w1, w2, w3, w4, w5, acc_sh, den_sh,
          a0, a1, a2, a3, a4, a5, g0, g1, g2, g3, g4, g5,
          s0, s1, s2, s3, s4, s5, stg):
        cid = lax.axis_index("c")
        sid = lax.axis_index("s")
        idxb = [i0, i1, i2, i3, i4, i5]
        rows = [r0, r1, r2, r3, r4, r5]
        wden = [w0, w1, w2, w3, w4, w5]
        isem = [a0, a1, a2, a3, a4, a5]
        gsem = [g0, g1, g2, g3, g4, g5]
        ssem = [s0, s1, s2, s3, s4, s5]
        tile = pl.ds(sid * RPT, RPT)

        copies = [
            (z64_hbm.at[tile], acc_sh.at[tile]),
            (z16_hbm.at[tile], den_sh.at[tile]),
            (el_hbm, el_v),
            (er_hbm, er_v),
            (bias_hbm, bias_v),
        ]
        descs = [pltpu.async_copy(a, b, stg) for a, b in copies]
        for d in descs:
            d.wait()
        plsc.subcore_barrier()

        def idx_start(m, b):
            pltpu.async_copy(idx_hbm.at[sid, m], idxb[b], isem[b])

        def idx_wait(m, b):
            pltpu.make_async_copy(idx_hbm.at[sid, m], idxb[b], isem[b]).wait()

        def gather_start(b):
            pltpu.async_copy(feat2_hbm.at[cid].at[idxb[b].at[0]],
                             rows[b], gsem[b])

        def gather_wait(b):
            pltpu.make_async_copy(feat2_hbm.at[cid].at[idxb[b].at[0]],
                                  rows[b], gsem[b]).wait()

        def scat_start(b):
            pltpu.async_copy(rows[b], acc_sh.at[idxb[b].at[1]], ssem[b],
                             add=True)
            pltpu.async_copy(wden[b], den_sh.at[idxb[b].at[1]], ssem[b],
                             add=True)

        def scat_wait(b):
            pltpu.make_async_copy(rows[b], acc_sh.at[idxb[b].at[1]],
                                  ssem[b]).wait()
            pltpu.make_async_copy(wden[b], den_sh.at[idxb[b].at[1]],
                                  ssem[b]).wait()

        def process(b):
            @plsc.parallel_loop(0, C, step=LANES)
            def _grp(g):
                si = idxb[b][0, pl.ds(g, LANES)]
                di = idxb[b][1, pl.ds(g, LANES)]
                e = plsc.load_gather(el_v, [si]) + plsc.load_gather(er_v, [di])
                e = jnp.where(e >= 0.0, e, 0.2 * e)
                wv = jnp.exp(e)
                for rr in range(LANES):
                    wb = jnp.broadcast_to(wv[rr], (LANES,))
                    wden[b][g + rr, :] = wb
                    for q in range(DG // LANES):
                        sl = pl.ds(q * LANES, LANES)
                        rows[b][g + rr, sl] = rows[b][g + rr, sl] * wb

        idx_start(0, 0)
        idx_start(1, 1)
        idx_start(2, 2)
        idx_wait(0, 0)
        gather_start(0)
        idx_wait(1, 1)
        gather_start(1)
        for s in range(NBUF):
            if s >= 3:
                scat_wait(s - 3)
            idx_start(s + 3, (s + 3) % NBUF)
            idx_wait(s + 2, (s + 2) % NBUF)
            gather_start((s + 2) % NBUF)
            gather_wait(s)
            process(s)
            scat_start(s)

        @pl.loop(NBUF, NCH, step=NBUF)
        def _steady(j):
            for off in range(NBUF):
                m = j + off
                bb3 = (off + 3) % NBUF
                bb2 = (off + 2) % NBUF
                scat_wait(bb3)

                @pl.when(m + 3 < NCH)
                def _():
                    idx_start(m + 3, bb3)

                @pl.when(m + 2 < NCH)
                def _():
                    idx_wait(m + 2, bb2)
                    gather_start(bb2)

                gather_wait(off)
                process(off)
                scat_start(off)

        scat_wait((NCH - 3) % NBUF)
        scat_wait((NCH - 2) % NBUF)
        scat_wait((NCH - 1) % NBUF)

        plsc.subcore_barrier()

        bias_regs = [bias_v[pl.ds(cid * DG + q * LANES, LANES)]
                     for q in range(DG // LANES)]

        @pl.loop(0, RPT, step=FB)
        def _fin(f):
            rb = sid * RPT + f
            pltpu.sync_copy(acc_sh.at[pl.ds(rb, FB)], r0.at[pl.ds(0, FB)])
            pltpu.sync_copy(den_sh.at[pl.ds(rb, FB)], w0.at[pl.ds(0, FB)])

            @pl.loop(0, FB)
            def _r(r):
                db = w0[r, pl.ds(0, LANES)]
                ok = db > 0.0
                for q in range(DG // LANES):
                    sl = pl.ds(q * LANES, LANES)
                    v = r0[r, sl]
                    r0[r, sl] = jnp.where(ok, v / db, 0.0) + bias_regs[q]

            pltpu.sync_copy(r0.at[pl.ds(0, FB)], out_hbm.at[cid, pl.ds(rb, FB)])

    return k(feat2, el, er, idx4, z64, z16, bias)


def kernel(x, edge_index, W, attn_l, attn_r, bias):
    feat2, el, er = _tc_project(x, W, attn_l, attn_r)
    ei3 = edge_index.reshape(2, NSUB, EPT)
    pad = jnp.full((2, NSUB, EPTP - EPT), N, jnp.int32)
    ei4 = jnp.concatenate([ei3, pad], axis=2)
    idx4 = ei4.reshape(2, NSUB, NCH, C).transpose(1, 2, 0, 3)
    z64 = jnp.zeros((NP, DG), jnp.float32)
    z16 = jnp.zeros((NP, DDEN), jnp.float32)
    halves = _sc_edge_aggregate(feat2, el, er, idx4, z64, z16, bias)
    return jnp.concatenate([halves[0, :N], halves[1, :N]], axis=1)
